# baseline (speedup 1.0000x reference)
import functools

import jax
import jax.numpy as jnp
from jax import lax
from jax.experimental import pallas as pl
from jax.experimental.pallas import tpu as pltpu
from jax.experimental.pallas import tpu_sc as plsc

N = 10000
D = 128
E = 320000

NC = 2
NS = 16
CHUNK = 128
Q = 160
TOTCH = NS * Q
EP = TOTCH * CHUNK
NPAD = 10240
RPT = NPAD // NS
ZCH = RPT // CHUNK

NBUF = 2
DWH = 8

_mesh = plsc.VectorSubcoreMesh(
    core_axis_name="c", subcore_axis_name="s", num_cores=NC, num_subcores=NS
)


@functools.partial(
    pl.kernel,
    out_type=jax.ShapeDtypeStruct((NPAD, D), jnp.float32),
    mesh=_mesh,
    scratch_types=[
        pltpu.VMEM((4, DWH, CHUNK), jnp.int32),
        pltpu.VMEM((2, DWH, CHUNK), jnp.int32),
        [pltpu.VMEM((CHUNK, D), jnp.float32) for _ in range(NBUF)],
        [pltpu.SemaphoreType.DMA for _ in range(NBUF)],
        pltpu.SemaphoreType.DMA,
        pltpu.SemaphoreType.DMA,
        pltpu.SemaphoreType.DMA,
        pltpu.VMEM_SHARED((NPAD, D), jnp.float32),
    ],
)
def _sc_agg(h_hbm, src_hbm, dst_hbm, out_hbm, src_w, dst_w, bufs, gsems, psem, ssem, dsem, acc):
    c = lax.axis_index("c")
    s = lax.axis_index("s")

    @pl.when(c == 0)
    def _():
        row0 = s * RPT
        base = s * Q

        pltpu.async_copy(src_hbm.at[pl.ds(base, DWH)], src_w.at[0], ssem)
        pltpu.async_copy(dst_hbm.at[pl.ds(base, DWH)], dst_w.at[0], dsem)

        @pl.loop(0, CHUNK)
        def _(i):
            for l in range(D // 16):
                bufs[0][i, pl.ds(l * 16, 16)] = jnp.zeros((16,), jnp.float32)

        for z in range(ZCH):
            pltpu.async_copy(bufs[0], acc.at[pl.ds(row0 + z * CHUNK, CHUNK)], psem)
        for z in range(ZCH):
            pltpu.make_async_copy(
                bufs[0], acc.at[pl.ds(row0 + z * CHUNK, CHUNK)], psem
            ).wait()
        plsc.subcore_barrier()

        pltpu.make_async_copy(src_hbm.at[pl.ds(base, DWH)], src_w.at[0], ssem).wait()
        pltpu.async_copy(src_hbm.at[pl.ds(base + DWH, DWH)], src_w.at[1], ssem)
        for k in range(NBUF):
            pltpu.async_copy(h_hbm.at[src_w.at[0, k]], bufs[k], gsems[k])

        @pl.loop(0, Q, step=NBUF)
        def _(j):
            for k in range(NBUF):
                jj = j + k
                if k == 0:
                    m = jj // DWH

                    @pl.when(lax.rem(jj, DWH) == 0)
                    def _():
                        pltpu.make_async_copy(
                            dst_hbm.at[pl.ds(base + m * DWH, DWH)],
                            dst_w.at[lax.rem(m, 2)],
                            dsem,
                        ).wait()

                        @pl.when((m + 1) * DWH < Q)
                        def _():
                            pltpu.async_copy(
                                dst_hbm.at[pl.ds(base + (m + 1) * DWH, DWH)],
                                dst_w.at[lax.rem(m + 1, 2)],
                                dsem,
                            )
                            pltpu.make_async_copy(
                                src_hbm.at[pl.ds(base + (m + 1) * DWH, DWH)],
                                src_w.at[lax.rem(m + 1, 4)],
                                ssem,
                            ).wait()

                            @pl.when((m + 2) * DWH < Q)
                            def _():
                                pltpu.async_copy(
                                    src_hbm.at[pl.ds(base + (m + 2) * DWH, DWH)],
                                    src_w.at[lax.rem(m + 2, 4)],
                                    ssem,
                                )

                pltpu.make_async_copy(
                    h_hbm.at[src_w.at[lax.rem(jj // DWH, 4), lax.rem(jj, DWH)]],
                    bufs[k],
                    gsems[k],
                ).wait()
                pltpu.sync_copy(
                    bufs[k],
                    acc.at[dst_w.at[lax.rem(jj // DWH, 2), lax.rem(jj, DWH)]],
                    add=True,
                )

                @pl.when(jj + NBUF < Q)
                def _():
                    jn = jj + NBUF
                    pltpu.async_copy(
                        h_hbm.at[src_w.at[lax.rem(jn // DWH, 4), lax.rem(jn, DWH)]],
                        bufs[k],
                        gsems[k],
                    )

        plsc.subcore_barrier()

        for z in range(ZCH):
            k = z % NBUF
            if z >= NBUF:
                pltpu.make_async_copy(
                    bufs[k],
                    out_hbm.at[pl.ds(row0 + (z - NBUF) * CHUNK, CHUNK)],
                    gsems[k],
                ).wait()
            pltpu.sync_copy(acc.at[pl.ds(row0 + z * CHUNK, CHUNK)], bufs[k])
            pltpu.async_copy(
                bufs[k], out_hbm.at[pl.ds(row0 + z * CHUNK, CHUNK)], gsems[k]
            )
        for z in range(max(ZCH - NBUF, 0), ZCH):
            k = z % NBUF
            pltpu.make_async_copy(
                bufs[k], out_hbm.at[pl.ds(row0 + z * CHUNK, CHUNK)], gsems[k]
            ).wait()


_BLK = 400


def _tc_body(x_ref, p_ref, w_ref, b_ref, o_ref):
    hin = x_ref[...] + p_ref[...]
    acc = lax.dot_general(
        hin,
        w_ref[...],
        (((1,), (0,)), ((), ())),
        preferred_element_type=jnp.float32,
        precision=lax.Precision.HIGHEST,
    )
    o_ref[...] = jnp.maximum(acc + b_ref[...], 0.0)


def _tc_layer(h, p, W, b):
    return pl.pallas_call(
        _tc_body,
        grid=(N // _BLK,),
        in_specs=[
            pl.BlockSpec((_BLK, D), lambda i: (i, 0)),
            pl.BlockSpec((_BLK, D), lambda i: (i, 0)),
            pl.BlockSpec((D, D), lambda i: (0, 0)),
            pl.BlockSpec((1, D), lambda i: (0, 0)),
        ],
        out_specs=pl.BlockSpec((_BLK, D), lambda i: (i, 0)),
        out_shape=jax.ShapeDtypeStruct((N, D), jnp.float32),
    )(h, p, W, b)


def kernel(x, edge_index, W1, b1, W2, b2):
    pad = EP - E
    src = jnp.concatenate([edge_index[0], jnp.zeros((pad,), jnp.int32)])
    trash = N + jnp.arange(pad, dtype=jnp.int32) % (NPAD - N)
    dst = jnp.concatenate([edge_index[1], trash])
    src = src.reshape(TOTCH, CHUNK)
    dst = dst.reshape(TOTCH, CHUNK)
    b1r = b1.reshape(1, D)
    b2r = b2.reshape(1, D)

    p1 = _sc_agg(x, src, dst)
    h1 = _tc_layer(x, p1, W1, b1r)
    p2 = _sc_agg(h1, src, dst)
    h2 = _tc_layer(h1, p2, W2, b2r)
    return h2


# speedup vs baseline: 2.5973x; 2.5973x over previous
import functools

import jax
import jax.numpy as jnp
from jax import lax
from jax.experimental import pallas as pl
from jax.experimental.pallas import tpu as pltpu
from jax.experimental.pallas import tpu_sc as plsc

N = 10000
D = 128
E = 320000

NC = 2
NS = 16
NW = NC * NS
CHUNK = 128
Q = 80
TOTCH = NW * Q
EP = TOTCH * CHUNK
NPAD = 10240
RPT = NPAD // NS
ZCH = RPT // CHUNK
TRPT = NPAD // NS
TZC = TRPT // CHUNK

NBUF = 2

_mesh = plsc.VectorSubcoreMesh(
    core_axis_name="c", subcore_axis_name="s", num_cores=NC, num_subcores=NS
)


@functools.partial(
    pl.kernel,
    out_type=jax.ShapeDtypeStruct((EP, D), jnp.float32),
    mesh=_mesh,
    scratch_types=[
        pltpu.VMEM((Q, CHUNK), jnp.int32),
        [pltpu.VMEM((CHUNK, D), jnp.float32) for _ in range(NBUF)],
        [pltpu.SemaphoreType.DMA for _ in range(NBUF)],
        [pltpu.SemaphoreType.DMA for _ in range(NBUF)],
        pltpu.SemaphoreType.DMA,
        pltpu.VMEM_SHARED((NPAD, D), jnp.float32),
    ],
)
def _sc_stage(h_hbm, src_hbm, stage_hbm, src_v, bufs, gsems, wsems, psem, tbl):
    c = lax.axis_index("c")
    s = lax.axis_index("s")
    wid = s * NC + c
    trow0 = s * TRPT

    pltpu.async_copy(src_hbm.at[wid], src_v, psem)
    for z in range(TZC):
        r0 = trow0 + z * CHUNK
        pltpu.sync_copy(h_hbm.at[pl.ds(r0, CHUNK)], bufs[0])
        pltpu.sync_copy(bufs[0], tbl.at[pl.ds(r0, CHUNK)])
    pltpu.make_async_copy(src_hbm.at[wid], src_v, psem).wait()
    plsc.subcore_barrier()

    for k in range(NBUF):
        pltpu.async_copy(tbl.at[src_v.at[k]], bufs[k], gsems[k])

    @pl.loop(0, Q, step=NBUF)
    def _(j):
        for k in range(NBUF):
            jj = j + k
            pltpu.make_async_copy(tbl.at[src_v.at[jj]], bufs[k], gsems[k]).wait()
            pltpu.async_copy(
                bufs[k],
                stage_hbm.at[pl.ds((wid * Q + jj) * CHUNK, CHUNK)],
                wsems[k],
            )

            @pl.when(jj + NBUF < Q)
            def _():
                pltpu.make_async_copy(
                    bufs[k],
                    stage_hbm.at[pl.ds((wid * Q + jj) * CHUNK, CHUNK)],
                    wsems[k],
                ).wait()
                pltpu.async_copy(tbl.at[src_v.at[jj + NBUF]], bufs[k], gsems[k])

    for k in range(NBUF):
        jj = Q - NBUF + k
        pltpu.make_async_copy(
            bufs[k],
            stage_hbm.at[pl.ds((wid * Q + jj) * CHUNK, CHUNK)],
            wsems[k],
        ).wait()


@functools.partial(
    pl.kernel,
    out_type=jax.ShapeDtypeStruct((NC * NPAD, D), jnp.float32),
    mesh=_mesh,
    scratch_types=[
        pltpu.VMEM((Q, CHUNK), jnp.int32),
        [pltpu.VMEM((CHUNK, D), jnp.float32) for _ in range(NBUF)],
        [pltpu.SemaphoreType.DMA for _ in range(NBUF)],
        pltpu.SemaphoreType.DMA,
        pltpu.VMEM_SHARED((NPAD, D), jnp.float32),
    ],
)
def _sc_scat(stage_hbm, dst_hbm, out_hbm, dst_v, bufs, rsems, psem, acc):
    c = lax.axis_index("c")
    s = lax.axis_index("s")
    wid = s * NC + c
    row0 = s * RPT

    pltpu.async_copy(dst_hbm.at[wid], dst_v, psem)

    @pl.loop(0, CHUNK)
    def _(i):
        for l in range(D // 16):
            bufs[0][i, pl.ds(l * 16, 16)] = jnp.zeros((16,), jnp.float32)

    for z in range(ZCH):
        pltpu.async_copy(bufs[0], acc.at[pl.ds(row0 + z * CHUNK, CHUNK)], psem)
    for z in range(ZCH):
        pltpu.make_async_copy(
            bufs[0], acc.at[pl.ds(row0 + z * CHUNK, CHUNK)], psem
        ).wait()
    pltpu.make_async_copy(dst_hbm.at[wid], dst_v, psem).wait()
    plsc.subcore_barrier()

    for k in range(NBUF):
        pltpu.async_copy(
            stage_hbm.at[pl.ds((wid * Q + k) * CHUNK, CHUNK)], bufs[k], rsems[k]
        )

    @pl.loop(0, Q, step=NBUF)
    def _(j):
        for k in range(NBUF):
            jj = j + k
            pltpu.make_async_copy(
                stage_hbm.at[pl.ds((wid * Q + jj) * CHUNK, CHUNK)], bufs[k], rsems[k]
            ).wait()
            pltpu.sync_copy(bufs[k], acc.at[dst_v.at[jj]], add=True)

            @pl.when(jj + NBUF < Q)
            def _():
                pltpu.async_copy(
                    stage_hbm.at[pl.ds((wid * Q + jj + NBUF) * CHUNK, CHUNK)],
                    bufs[k],
                    rsems[k],
                )

    plsc.subcore_barrier()

    for z in range(ZCH):
        k = z % NBUF
        if z >= NBUF:
            pltpu.make_async_copy(
                bufs[k],
                out_hbm.at[pl.ds(c * NPAD + row0 + (z - NBUF) * CHUNK, CHUNK)],
                rsems[k],
            ).wait()
        pltpu.sync_copy(acc.at[pl.ds(row0 + z * CHUNK, CHUNK)], bufs[k])
        pltpu.async_copy(
            bufs[k], out_hbm.at[pl.ds(c * NPAD + row0 + z * CHUNK, CHUNK)], rsems[k]
        )
    for z in range(max(ZCH - NBUF, 0), ZCH):
        k = z % NBUF
        pltpu.make_async_copy(
            bufs[k], out_hbm.at[pl.ds(c * NPAD + row0 + z * CHUNK, CHUNK)], rsems[k]
        ).wait()


_BLK = 400


def _tc_body(x_ref, p_ref, w_ref, b_ref, o_ref):
    hin = x_ref[...] + p_ref[0] + p_ref[1]
    acc = lax.dot_general(
        hin,
        w_ref[...],
        (((1,), (0,)), ((), ())),
        preferred_element_type=jnp.float32,
        precision=lax.Precision.HIGHEST,
    )
    o_ref[...] = jnp.maximum(acc + b_ref[...], 0.0)


def _tc_layer(h, p, W, b):
    return pl.pallas_call(
        _tc_body,
        grid=(N // _BLK,),
        in_specs=[
            pl.BlockSpec((_BLK, D), lambda i: (i, 0)),
            pl.BlockSpec((2, _BLK, D), lambda i: (0, i, 0)),
            pl.BlockSpec((D, D), lambda i: (0, 0)),
            pl.BlockSpec((1, D), lambda i: (0, 0)),
        ],
        out_specs=pl.BlockSpec((_BLK, D), lambda i: (i, 0)),
        out_shape=jax.ShapeDtypeStruct((N, D), jnp.float32),
    )(h, p, W, b)


def _agg(h, src, dst):
    hp = jnp.pad(h, ((0, NPAD - N), (0, 0)))
    stage = _sc_stage(hp, src)
    return _sc_scat(stage, dst).reshape(NC, NPAD, D)


def kernel(x, edge_index, W1, b1, W2, b2):
    pad = EP - E
    src = jnp.concatenate([edge_index[0], jnp.zeros((pad,), jnp.int32)])
    trash = N + jnp.arange(pad, dtype=jnp.int32) % (NPAD - N)
    dst = jnp.concatenate([edge_index[1], trash])
    src = src.reshape(NW, Q, CHUNK)
    dst = dst.reshape(NW, Q, CHUNK)
    b1r = b1.reshape(1, D)
    b2r = b2.reshape(1, D)

    p1 = _agg(x, src, dst)
    h1 = _tc_layer(x, p1, W1, b1r)
    p2 = _agg(h1, src, dst)
    h2 = _tc_layer(h1, p2, W2, b2r)
    return h2


# speedup vs baseline: 2.6723x; 1.0289x over previous
import functools

import jax
import jax.numpy as jnp
from jax import lax
from jax.experimental import pallas as pl
from jax.experimental.pallas import tpu as pltpu
from jax.experimental.pallas import tpu_sc as plsc

N = 10000
D = 128
E = 320000

NC = 2
NS = 16
NW = NC * NS
CHUNK = 128
Q = 80
TOTCH = NW * Q
EP = TOTCH * CHUNK
NPAD = 10240
RPT = NPAD // NS
ZCH = RPT // CHUNK
TRPT = NPAD // NS
TZC = TRPT // CHUNK

NBUF = 2

_mesh = plsc.VectorSubcoreMesh(
    core_axis_name="c", subcore_axis_name="s", num_cores=NC, num_subcores=NS
)


@functools.partial(
    pl.kernel,
    out_type=jax.ShapeDtypeStruct((EP, D), jnp.float32),
    mesh=_mesh,
    scratch_types=[
        pltpu.VMEM((Q, CHUNK), jnp.int32),
        [pltpu.VMEM((CHUNK, D), jnp.float32) for _ in range(NBUF)],
        [pltpu.SemaphoreType.DMA for _ in range(NBUF)],
        [pltpu.SemaphoreType.DMA for _ in range(NBUF)],
        pltpu.SemaphoreType.DMA,
        pltpu.VMEM_SHARED((NPAD, D), jnp.float32),
    ],
)
def _sc_stage(h_hbm, src_hbm, stage_hbm, src_v, bufs, gsems, wsems, psem, tbl):
    c = lax.axis_index("c")
    s = lax.axis_index("s")
    wid = s * NC + c
    trow0 = s * TRPT

    pltpu.async_copy(src_hbm.at[wid], src_v, psem)
    trow0c = pl.multiple_of(jnp.minimum(trow0, N - TRPT), 8)
    for z in range(TZC):
        k = z % NBUF
        r0 = trow0c + z * CHUNK
        if z >= NBUF:
            rp = trow0c + (z - NBUF) * CHUNK
            pltpu.make_async_copy(bufs[k], tbl.at[pl.ds(rp, CHUNK)], gsems[k]).wait()
        pltpu.sync_copy(h_hbm.at[pl.ds(r0, CHUNK)], bufs[k])
        pltpu.async_copy(bufs[k], tbl.at[pl.ds(r0, CHUNK)], gsems[k])
    for z in range(max(TZC - NBUF, 0), TZC):
        k = z % NBUF
        r0 = trow0c + z * CHUNK
        pltpu.make_async_copy(bufs[k], tbl.at[pl.ds(r0, CHUNK)], gsems[k]).wait()
    pltpu.make_async_copy(src_hbm.at[wid], src_v, psem).wait()
    plsc.subcore_barrier()

    for k in range(NBUF):
        pltpu.async_copy(tbl.at[src_v.at[k]], bufs[k], gsems[k])

    @pl.loop(0, Q, step=NBUF)
    def _(j):
        for k in range(NBUF):
            jj = j + k
            pltpu.make_async_copy(tbl.at[src_v.at[jj]], bufs[k], gsems[k]).wait()
            pltpu.async_copy(
                bufs[k],
                stage_hbm.at[pl.ds((wid * Q + jj) * CHUNK, CHUNK)],
                wsems[k],
            )

            @pl.when(jj + NBUF < Q)
            def _():
                pltpu.make_async_copy(
                    bufs[k],
                    stage_hbm.at[pl.ds((wid * Q + jj) * CHUNK, CHUNK)],
                    wsems[k],
                ).wait()
                pltpu.async_copy(tbl.at[src_v.at[jj + NBUF]], bufs[k], gsems[k])

    for k in range(NBUF):
        jj = Q - NBUF + k
        pltpu.make_async_copy(
            bufs[k],
            stage_hbm.at[pl.ds((wid * Q + jj) * CHUNK, CHUNK)],
            wsems[k],
        ).wait()


@functools.partial(
    pl.kernel,
    out_type=jax.ShapeDtypeStruct((NC * NPAD, D), jnp.float32),
    mesh=_mesh,
    scratch_types=[
        pltpu.VMEM((Q, CHUNK), jnp.int32),
        [pltpu.VMEM((CHUNK, D), jnp.float32) for _ in range(NBUF)],
        [pltpu.SemaphoreType.DMA for _ in range(NBUF)],
        pltpu.SemaphoreType.DMA,
        pltpu.VMEM_SHARED((NPAD, D), jnp.float32),
    ],
)
def _sc_scat(stage_hbm, dst_hbm, out_hbm, dst_v, bufs, rsems, psem, acc):
    c = lax.axis_index("c")
    s = lax.axis_index("s")
    wid = s * NC + c
    row0 = s * RPT

    pltpu.async_copy(dst_hbm.at[wid], dst_v, psem)

    @pl.loop(0, CHUNK)
    def _(i):
        for l in range(D // 16):
            bufs[0][i, pl.ds(l * 16, 16)] = jnp.zeros((16,), jnp.float32)

    for z in range(ZCH):
        pltpu.async_copy(bufs[0], acc.at[pl.ds(row0 + z * CHUNK, CHUNK)], psem)
    for z in range(ZCH):
        pltpu.make_async_copy(
            bufs[0], acc.at[pl.ds(row0 + z * CHUNK, CHUNK)], psem
        ).wait()
    pltpu.make_async_copy(dst_hbm.at[wid], dst_v, psem).wait()
    plsc.subcore_barrier()

    for k in range(NBUF):
        pltpu.async_copy(
            stage_hbm.at[pl.ds((wid * Q + k) * CHUNK, CHUNK)], bufs[k], rsems[k]
        )

    @pl.loop(0, Q, step=NBUF)
    def _(j):
        for k in range(NBUF):
            jj = j + k
            pltpu.make_async_copy(
                stage_hbm.at[pl.ds((wid * Q + jj) * CHUNK, CHUNK)], bufs[k], rsems[k]
            ).wait()
            pltpu.sync_copy(bufs[k], acc.at[dst_v.at[jj]], add=True)

            @pl.when(jj + NBUF < Q)
            def _():
                pltpu.async_copy(
                    stage_hbm.at[pl.ds((wid * Q + jj + NBUF) * CHUNK, CHUNK)],
                    bufs[k],
                    rsems[k],
                )

    plsc.subcore_barrier()

    for z in range(ZCH):
        k = z % NBUF
        if z >= NBUF:
            pltpu.make_async_copy(
                bufs[k],
                out_hbm.at[pl.ds(c * NPAD + row0 + (z - NBUF) * CHUNK, CHUNK)],
                rsems[k],
            ).wait()
        pltpu.sync_copy(acc.at[pl.ds(row0 + z * CHUNK, CHUNK)], bufs[k])
        pltpu.async_copy(
            bufs[k], out_hbm.at[pl.ds(c * NPAD + row0 + z * CHUNK, CHUNK)], rsems[k]
        )
    for z in range(max(ZCH - NBUF, 0), ZCH):
        k = z % NBUF
        pltpu.make_async_copy(
            bufs[k], out_hbm.at[pl.ds(c * NPAD + row0 + z * CHUNK, CHUNK)], rsems[k]
        ).wait()


_BLK = 400


def _tc_body(x_ref, p_ref, w_ref, b_ref, o_ref):
    hin = x_ref[...] + p_ref[0] + p_ref[1]
    acc = lax.dot_general(
        hin,
        w_ref[...],
        (((1,), (0,)), ((), ())),
        preferred_element_type=jnp.float32,
        precision=lax.Precision.HIGHEST,
    )
    o_ref[...] = jnp.maximum(acc + b_ref[...], 0.0)


def _tc_layer(h, p, W, b):
    return pl.pallas_call(
        _tc_body,
        grid=(N // _BLK,),
        in_specs=[
            pl.BlockSpec((_BLK, D), lambda i: (i, 0)),
            pl.BlockSpec((2, _BLK, D), lambda i: (0, i, 0)),
            pl.BlockSpec((D, D), lambda i: (0, 0)),
            pl.BlockSpec((1, D), lambda i: (0, 0)),
        ],
        out_specs=pl.BlockSpec((_BLK, D), lambda i: (i, 0)),
        out_shape=jax.ShapeDtypeStruct((N, D), jnp.float32),
    )(h, p, W, b)


def _agg(h, src, dst):
    stage = _sc_stage(h, src)
    return _sc_scat(stage, dst).reshape(NC, NPAD, D)


def kernel(x, edge_index, W1, b1, W2, b2):
    pad = EP - E
    src = jnp.concatenate([edge_index[0], jnp.zeros((pad,), jnp.int32)])
    trash = N + jnp.arange(pad, dtype=jnp.int32) % (NPAD - N)
    dst = jnp.concatenate([edge_index[1], trash])
    src = src.reshape(NW, Q, CHUNK)
    dst = dst.reshape(NW, Q, CHUNK)
    b1r = b1.reshape(1, D)
    b2r = b2.reshape(1, D)

    p1 = _agg(x, src, dst)
    h1 = _tc_layer(x, p1, W1, b1r)
    p2 = _agg(h1, src, dst)
    h2 = _tc_layer(h1, p2, W2, b2r)
    return h2


# speedup vs baseline: 2.6972x; 1.0093x over previous
import functools

import jax
import jax.numpy as jnp
from jax import lax
from jax.experimental import pallas as pl
from jax.experimental.pallas import tpu as pltpu
from jax.experimental.pallas import tpu_sc as plsc

N = 10000
D = 128
E = 320000

NC = 2
NS = 16
NW = NC * NS
CHUNK = 128
Q = 80
TOTCH = NW * Q
EP = TOTCH * CHUNK
NPAD = 10240
RPT = NPAD // NS
ZCH = RPT // CHUNK
TRPT = NPAD // NS
TZC = TRPT // CHUNK

NBUF = 2

_mesh = plsc.VectorSubcoreMesh(
    core_axis_name="c", subcore_axis_name="s", num_cores=NC, num_subcores=NS
)


@functools.partial(
    pl.kernel,
    out_type=jax.ShapeDtypeStruct((EP, D), jnp.float32),
    mesh=_mesh,
    scratch_types=[
        pltpu.VMEM((Q, CHUNK), jnp.int32),
        [pltpu.VMEM((CHUNK, D), jnp.float32) for _ in range(NBUF)],
        [pltpu.SemaphoreType.DMA for _ in range(NBUF)],
        [pltpu.SemaphoreType.DMA for _ in range(NBUF)],
        pltpu.SemaphoreType.DMA,
        pltpu.VMEM_SHARED((NPAD, D), jnp.float32),
    ],
)
def _sc_stage(h_hbm, src_hbm, stage_hbm, src_v, bufs, gsems, wsems, psem, tbl):
    c = lax.axis_index("c")
    s = lax.axis_index("s")
    wid = s * NC + c
    trow0 = s * TRPT

    pltpu.async_copy(src_hbm.at[wid], src_v, psem)
    trow0c = pl.multiple_of(jnp.minimum(trow0, N - TRPT), 8)
    for z in range(TZC):
        k = z % NBUF
        r0 = trow0c + z * CHUNK
        if z >= NBUF:
            rp = trow0c + (z - NBUF) * CHUNK
            pltpu.make_async_copy(bufs[k], tbl.at[pl.ds(rp, CHUNK)], gsems[k]).wait()
        pltpu.sync_copy(h_hbm.at[pl.ds(r0, CHUNK)], bufs[k])
        pltpu.async_copy(bufs[k], tbl.at[pl.ds(r0, CHUNK)], gsems[k])
    for z in range(max(TZC - NBUF, 0), TZC):
        k = z % NBUF
        r0 = trow0c + z * CHUNK
        pltpu.make_async_copy(bufs[k], tbl.at[pl.ds(r0, CHUNK)], gsems[k]).wait()
    pltpu.make_async_copy(src_hbm.at[wid], src_v, psem).wait()
    plsc.subcore_barrier()

    for k in range(NBUF):
        pltpu.async_copy(tbl.at[src_v.at[k]], bufs[k], gsems[k])

    @pl.loop(0, Q, step=NBUF)
    def _(j):
        for k in range(NBUF):
            jj = j + k
            pltpu.make_async_copy(tbl.at[src_v.at[jj]], bufs[k], gsems[k]).wait()
            pltpu.async_copy(
                bufs[k],
                stage_hbm.at[pl.ds((wid * Q + jj) * CHUNK, CHUNK)],
                wsems[k],
            )

            @pl.when(jj + NBUF < Q)
            def _():
                pltpu.make_async_copy(
                    bufs[k],
                    stage_hbm.at[pl.ds((wid * Q + jj) * CHUNK, CHUNK)],
                    wsems[k],
                ).wait()
                pltpu.async_copy(tbl.at[src_v.at[jj + NBUF]], bufs[k], gsems[k])

    for k in range(NBUF):
        jj = Q - NBUF + k
        pltpu.make_async_copy(
            bufs[k],
            stage_hbm.at[pl.ds((wid * Q + jj) * CHUNK, CHUNK)],
            wsems[k],
        ).wait()


@functools.partial(
    pl.kernel,
    out_type=jax.ShapeDtypeStruct((NC * NPAD, D), jnp.float32),
    mesh=_mesh,
    scratch_types=[
        pltpu.VMEM((Q, CHUNK), jnp.int32),
        [pltpu.VMEM((CHUNK, D), jnp.float32) for _ in range(NBUF)],
        [pltpu.SemaphoreType.DMA for _ in range(NBUF)],
        pltpu.SemaphoreType.DMA,
        pltpu.VMEM_SHARED((NPAD, D), jnp.float32),
    ],
)
def _sc_scat(stage_hbm, dst_hbm, out_hbm, dst_v, bufs, rsems, psem, acc):
    c = lax.axis_index("c")
    s = lax.axis_index("s")
    wid = s * NC + c
    row0 = s * RPT

    pltpu.async_copy(dst_hbm.at[wid], dst_v, psem)

    @pl.loop(0, CHUNK)
    def _(i):
        for l in range(D // 16):
            bufs[0][i, pl.ds(l * 16, 16)] = jnp.zeros((16,), jnp.float32)

    for z in range(ZCH):
        pltpu.async_copy(bufs[0], acc.at[pl.ds(row0 + z * CHUNK, CHUNK)], psem)
    for z in range(ZCH):
        pltpu.make_async_copy(
            bufs[0], acc.at[pl.ds(row0 + z * CHUNK, CHUNK)], psem
        ).wait()
    pltpu.make_async_copy(dst_hbm.at[wid], dst_v, psem).wait()
    plsc.subcore_barrier()

    for k in range(NBUF):
        pltpu.async_copy(
            stage_hbm.at[pl.ds((wid * Q + k) * CHUNK, CHUNK)], bufs[k], rsems[k]
        )

    @pl.loop(0, Q, step=NBUF)
    def _(j):
        for k in range(NBUF):
            jj = j + k
            pltpu.make_async_copy(
                stage_hbm.at[pl.ds((wid * Q + jj) * CHUNK, CHUNK)], bufs[k], rsems[k]
            ).wait()
            pltpu.sync_copy(bufs[k], acc.at[dst_v.at[jj]], add=True)

            @pl.when(jj + NBUF < Q)
            def _():
                pltpu.async_copy(
                    stage_hbm.at[pl.ds((wid * Q + jj + NBUF) * CHUNK, CHUNK)],
                    bufs[k],
                    rsems[k],
                )

    plsc.subcore_barrier()

    for z in range(ZCH):
        k = z % NBUF
        if z >= NBUF:
            pltpu.make_async_copy(
                bufs[k],
                out_hbm.at[pl.ds(c * NPAD + row0 + (z - NBUF) * CHUNK, CHUNK)],
                rsems[k],
            ).wait()
        pltpu.sync_copy(acc.at[pl.ds(row0 + z * CHUNK, CHUNK)], bufs[k])
        pltpu.async_copy(
            bufs[k], out_hbm.at[pl.ds(c * NPAD + row0 + z * CHUNK, CHUNK)], rsems[k]
        )
    for z in range(max(ZCH - NBUF, 0), ZCH):
        k = z % NBUF
        pltpu.make_async_copy(
            bufs[k], out_hbm.at[pl.ds(c * NPAD + row0 + z * CHUNK, CHUNK)], rsems[k]
        ).wait()


_BLK = 400


def _tc_body(x_ref, p_ref, w_ref, b_ref, o_ref):
    hin = x_ref[...] + p_ref[0] + p_ref[1]
    acc = lax.dot_general(
        hin,
        w_ref[...],
        (((1,), (0,)), ((), ())),
        preferred_element_type=jnp.float32,
        precision=lax.Precision.DEFAULT,
    )
    o_ref[...] = jnp.maximum(acc + b_ref[...], 0.0)


def _tc_layer(h, p, W, b):
    return pl.pallas_call(
        _tc_body,
        grid=(N // _BLK,),
        in_specs=[
            pl.BlockSpec((_BLK, D), lambda i: (i, 0)),
            pl.BlockSpec((2, _BLK, D), lambda i: (0, i, 0)),
            pl.BlockSpec((D, D), lambda i: (0, 0)),
            pl.BlockSpec((1, D), lambda i: (0, 0)),
        ],
        out_specs=pl.BlockSpec((_BLK, D), lambda i: (i, 0)),
        out_shape=jax.ShapeDtypeStruct((N, D), jnp.float32),
    )(h, p, W, b)


def _agg(h, src, dst):
    stage = _sc_stage(h, src)
    return _sc_scat(stage, dst).reshape(NC, NPAD, D)


def kernel(x, edge_index, W1, b1, W2, b2):
    pad = EP - E
    src = jnp.concatenate([edge_index[0], jnp.zeros((pad,), jnp.int32)])
    trash = N + jnp.arange(pad, dtype=jnp.int32) % (NPAD - N)
    dst = jnp.concatenate([edge_index[1], trash])
    src = src.reshape(NW, Q, CHUNK)
    dst = dst.reshape(NW, Q, CHUNK)
    b1r = b1.reshape(1, D)
    b2r = b2.reshape(1, D)

    p1 = _agg(x, src, dst)
    h1 = _tc_layer(x, p1, W1, b1r)
    p2 = _agg(h1, src, dst)
    h2 = _tc_layer(h1, p2, W2, b2r)
    return h2


# speedup vs baseline: 2.7973x; 1.0371x over previous
import functools

import jax
import jax.numpy as jnp
from jax import lax
from jax.experimental import pallas as pl
from jax.experimental.pallas import tpu as pltpu
from jax.experimental.pallas import tpu_sc as plsc

N = 10000
D = 128
E = 320000

NC = 2
NS = 16
NW = NC * NS
CHUNK = 128
Q = 80
TOTCH = NW * Q
EP = TOTCH * CHUNK
NPAD = 10240
RPT = NPAD // NS
ZCH = RPT // CHUNK
TRPT = NPAD // NS
TZC = TRPT // CHUNK

NBUF = 2

_mesh = plsc.VectorSubcoreMesh(
    core_axis_name="c", subcore_axis_name="s", num_cores=NC, num_subcores=NS
)


@functools.partial(
    pl.kernel,
    out_type=jax.ShapeDtypeStruct((EP, D), jnp.float32),
    mesh=_mesh,
    scratch_types=[
        pltpu.VMEM((Q, CHUNK), jnp.int32),
        [pltpu.VMEM((CHUNK, D), jnp.float32) for _ in range(NBUF)],
        [pltpu.SemaphoreType.DMA for _ in range(NBUF)],
        [pltpu.SemaphoreType.DMA for _ in range(NBUF)],
        pltpu.SemaphoreType.DMA,
        pltpu.VMEM_SHARED((NPAD, D), jnp.float32),
    ],
)
def _sc_stage(h_hbm, src_hbm, stage_hbm, src_v, bufs, gsems, wsems, psem, tbl):
    c = lax.axis_index("c")
    s = lax.axis_index("s")
    wid = s * NC + c
    trow0 = s * TRPT

    pltpu.async_copy(src_hbm.at[wid], src_v, psem)
    trow0c = pl.multiple_of(jnp.minimum(trow0, N - TRPT), 8)
    for z in range(TZC):
        k = z % NBUF
        r0 = trow0c + z * CHUNK
        if z >= NBUF:
            rp = trow0c + (z - NBUF) * CHUNK
            pltpu.make_async_copy(bufs[k], tbl.at[pl.ds(rp, CHUNK)], gsems[k]).wait()
        pltpu.sync_copy(h_hbm.at[pl.ds(r0, CHUNK)], bufs[k])
        pltpu.async_copy(bufs[k], tbl.at[pl.ds(r0, CHUNK)], gsems[k])
    for z in range(max(TZC - NBUF, 0), TZC):
        k = z % NBUF
        r0 = trow0c + z * CHUNK
        pltpu.make_async_copy(bufs[k], tbl.at[pl.ds(r0, CHUNK)], gsems[k]).wait()
    pltpu.make_async_copy(src_hbm.at[wid], src_v, psem).wait()
    plsc.subcore_barrier()

    for k in range(NBUF):
        pltpu.async_copy(tbl.at[src_v.at[k]], bufs[k], gsems[k])

    @pl.loop(0, Q, step=NBUF)
    def _(j):
        for k in range(NBUF):
            jj = j + k
            pltpu.make_async_copy(tbl.at[src_v.at[jj]], bufs[k], gsems[k]).wait()
            pltpu.async_copy(
                bufs[k],
                stage_hbm.at[pl.ds((wid * Q + jj) * CHUNK, CHUNK)],
                wsems[k],
            )

            @pl.when(jj + NBUF < Q)
            def _():
                pltpu.make_async_copy(
                    bufs[k],
                    stage_hbm.at[pl.ds((wid * Q + jj) * CHUNK, CHUNK)],
                    wsems[k],
                ).wait()
                pltpu.async_copy(tbl.at[src_v.at[jj + NBUF]], bufs[k], gsems[k])

    for k in range(NBUF):
        jj = Q - NBUF + k
        pltpu.make_async_copy(
            bufs[k],
            stage_hbm.at[pl.ds((wid * Q + jj) * CHUNK, CHUNK)],
            wsems[k],
        ).wait()


@functools.partial(
    pl.kernel,
    out_type=jax.ShapeDtypeStruct((NC * NPAD, D), jnp.float32),
    mesh=_mesh,
    scratch_types=[
        pltpu.VMEM((Q, CHUNK), jnp.int32),
        [pltpu.VMEM((CHUNK, D), jnp.float32) for _ in range(NBUF)],
        [pltpu.SemaphoreType.DMA for _ in range(NBUF)],
        pltpu.SemaphoreType.DMA,
        pltpu.VMEM_SHARED((NPAD, D), jnp.float32),
    ],
)
def _sc_scat(stage_hbm, dst_hbm, out_hbm, dst_v, bufs, rsems, psem, acc):
    c = lax.axis_index("c")
    s = lax.axis_index("s")
    wid = s * NC + c
    row0 = s * RPT

    pltpu.async_copy(dst_hbm.at[wid], dst_v, psem)

    @pl.loop(0, CHUNK)
    def _(i):
        for l in range(D // 16):
            bufs[0][i, pl.ds(l * 16, 16)] = jnp.zeros((16,), jnp.float32)

    for z in range(ZCH):
        pltpu.async_copy(bufs[0], acc.at[pl.ds(row0 + z * CHUNK, CHUNK)], psem)
    for z in range(ZCH):
        pltpu.make_async_copy(
            bufs[0], acc.at[pl.ds(row0 + z * CHUNK, CHUNK)], psem
        ).wait()
    pltpu.make_async_copy(dst_hbm.at[wid], dst_v, psem).wait()
    plsc.subcore_barrier()

    for k in range(NBUF):
        pltpu.async_copy(
            stage_hbm.at[pl.ds((wid * Q + k) * CHUNK, CHUNK)], bufs[k], rsems[k]
        )

    @pl.loop(0, Q, step=NBUF)
    def _(j):
        for k in range(NBUF):
            jj = j + k
            pltpu.make_async_copy(
                stage_hbm.at[pl.ds((wid * Q + jj) * CHUNK, CHUNK)], bufs[k], rsems[k]
            ).wait()
            pltpu.sync_copy(bufs[k], acc.at[dst_v.at[jj]], add=True)

            @pl.when(jj + NBUF < Q)
            def _():
                pltpu.async_copy(
                    stage_hbm.at[pl.ds((wid * Q + jj + NBUF) * CHUNK, CHUNK)],
                    bufs[k],
                    rsems[k],
                )

    plsc.subcore_barrier()

    for z in range(ZCH):
        k = z % NBUF
        if z >= NBUF:
            pltpu.make_async_copy(
                bufs[k],
                out_hbm.at[pl.ds(c * NPAD + row0 + (z - NBUF) * CHUNK, CHUNK)],
                rsems[k],
            ).wait()
        pltpu.sync_copy(acc.at[pl.ds(row0 + z * CHUNK, CHUNK)], bufs[k])
        pltpu.async_copy(
            bufs[k], out_hbm.at[pl.ds(c * NPAD + row0 + z * CHUNK, CHUNK)], rsems[k]
        )
    for z in range(max(ZCH - NBUF, 0), ZCH):
        k = z % NBUF
        pltpu.make_async_copy(
            bufs[k], out_hbm.at[pl.ds(c * NPAD + row0 + z * CHUNK, CHUNK)], rsems[k]
        ).wait()


_BLK = 1000


def _tc_body(x_ref, p_ref, w_ref, b_ref, o_ref):
    hin = x_ref[...] + p_ref[0] + p_ref[1]
    acc = lax.dot_general(
        hin,
        w_ref[...],
        (((1,), (0,)), ((), ())),
        preferred_element_type=jnp.float32,
        precision=lax.Precision.DEFAULT,
    )
    o_ref[...] = jnp.maximum(acc + b_ref[...], 0.0)


def _tc_layer(h, p, W, b):
    return pl.pallas_call(
        _tc_body,
        grid=(N // _BLK,),
        in_specs=[
            pl.BlockSpec((_BLK, D), lambda i: (i, 0)),
            pl.BlockSpec((2, _BLK, D), lambda i: (0, i, 0)),
            pl.BlockSpec((D, D), lambda i: (0, 0)),
            pl.BlockSpec((1, D), lambda i: (0, 0)),
        ],
        out_specs=pl.BlockSpec((_BLK, D), lambda i: (i, 0)),
        out_shape=jax.ShapeDtypeStruct((N, D), jnp.float32),
    )(h, p, W, b)


def _agg(h, src, dst):
    stage = _sc_stage(h, src)
    return _sc_scat(stage, dst).reshape(NC, NPAD, D)


def kernel(x, edge_index, W1, b1, W2, b2):
    pad = EP - E
    src = jnp.concatenate([edge_index[0], jnp.zeros((pad,), jnp.int32)])
    trash = N + jnp.arange(pad, dtype=jnp.int32) % (NPAD - N)
    dst = jnp.concatenate([edge_index[1], trash])
    src = src.reshape(NW, Q, CHUNK)
    dst = dst.reshape(NW, Q, CHUNK)
    b1r = b1.reshape(1, D)
    b2r = b2.reshape(1, D)

    p1 = _agg(x, src, dst)
    h1 = _tc_layer(x, p1, W1, b1r)
    p2 = _agg(h1, src, dst)
    h2 = _tc_layer(h1, p2, W2, b2r)
    return h2
